# Initial kernel scaffold; baseline (speedup 1.0000x reference)
import jax
import jax.numpy as jnp
from jax.experimental import pallas as pl


def kernel(x, norm_w, router_w, We1, be1, We2, be2, Ws1, bs1, Ws2, bs2):
    raise NotImplementedError("write your pallas kernel here")


# speedup vs baseline: 1.0447x; 1.0447x over previous
import functools

import jax
import jax.numpy as jnp
from jax import lax
from jax.experimental import pallas as pl
from jax.experimental.pallas import tpu as pltpu
from jax.experimental.pallas import tpu_sc as plsc

K_POOL = 4
NUM_EXPERTS = 8
D_IN = 3072
D_H = 1024
D_OUT = 2048

BM = 128
NT = 2048 // BM + NUM_EXPERTS - 1
NT = 24
P = NT * BM
NW = 32


def _gelu(h):
    return 0.5 * h * (1.0 + lax.erf(h * (2.0 ** -0.5)))


def _norm_router_shared_body(x_ref, nw_ref, rw_ref, ws1_ref, bs1_ref, ws2_ref,
                             bs2_ref, xn_ref, ti_ref, tw_ref, s_ref):
    x = x_ref[...]
    var = jnp.mean(x * x, axis=1, keepdims=True)
    xn = (x * lax.rsqrt(var + 1e-6)) * nw_ref[...]
    xn_ref[...] = xn

    logits = lax.dot_general(xn, rw_ref[...], (((1,), (1,)), ((), ())),
                             preferred_element_type=jnp.float32)
    m = jnp.max(logits, axis=1, keepdims=True)
    e = jnp.exp(logits - m)
    p = e / jnp.sum(e, axis=1, keepdims=True)
    idx = lax.broadcasted_iota(jnp.int32, (BM, NUM_EXPERTS), 1)
    m1 = jnp.max(p, axis=1, keepdims=True)
    i1 = jnp.min(jnp.where(p == m1, idx, NUM_EXPERTS), axis=1, keepdims=True)
    p2 = jnp.where(idx == i1, -1.0, p)
    m2 = jnp.max(p2, axis=1, keepdims=True)
    i2 = jnp.min(jnp.where(p2 == m2, idx, NUM_EXPERTS), axis=1, keepdims=True)
    denom = m1 + m2 + 1e-6
    ti_ref[...] = jnp.concatenate([i1, i2], axis=1)
    tw_ref[...] = jnp.concatenate([m1 / denom, m2 / denom], axis=1)

    h = lax.dot_general(xn, ws1_ref[...], (((1,), (1,)), ((), ())),
                        preferred_element_type=jnp.float32) + bs1_ref[...]
    h = _gelu(h)
    s_ref[...] = lax.dot_general(h, ws2_ref[...], (((1,), (1,)), ((), ())),
                                 preferred_element_type=jnp.float32) + bs2_ref[...]


def _norm_router_shared(flat, norm_w, router_w, Ws1, bs1, Ws2, bs2):
    ntok = flat.shape[0]
    grid = (ntok // BM,)
    return pl.pallas_call(
        _norm_router_shared_body,
        grid=grid,
        in_specs=[
            pl.BlockSpec((BM, D_IN), lambda i: (i, 0)),
            pl.BlockSpec((1, D_IN), lambda i: (0, 0)),
            pl.BlockSpec((NUM_EXPERTS, D_IN), lambda i: (0, 0)),
            pl.BlockSpec((D_H, D_IN), lambda i: (0, 0)),
            pl.BlockSpec((1, D_H), lambda i: (0, 0)),
            pl.BlockSpec((D_OUT, D_H), lambda i: (0, 0)),
            pl.BlockSpec((1, D_OUT), lambda i: (0, 0)),
        ],
        out_specs=[
            pl.BlockSpec((BM, D_IN), lambda i: (i, 0)),
            pl.BlockSpec((BM, 2), lambda i: (i, 0)),
            pl.BlockSpec((BM, 2), lambda i: (i, 0)),
            pl.BlockSpec((BM, D_OUT), lambda i: (i, 0)),
        ],
        out_shape=[
            jax.ShapeDtypeStruct((ntok, D_IN), jnp.float32),
            jax.ShapeDtypeStruct((ntok, 2), jnp.int32),
            jax.ShapeDtypeStruct((ntok, 2), jnp.float32),
            jax.ShapeDtypeStruct((ntok, D_OUT), jnp.float32),
        ],
    )(flat, norm_w.reshape(1, D_IN), router_w, Ws1, bs1.reshape(1, D_H),
      Ws2, bs2.reshape(1, D_OUT))


def _sc_gather_rows(xn, row_map):
    CH = 16
    rpw = P // NW
    mesh = plsc.VectorSubcoreMesh(core_axis_name="c", subcore_axis_name="s")

    @functools.partial(
        pl.kernel, mesh=mesh,
        out_type=jax.ShapeDtypeStruct((P, D_IN), jnp.float32),
        scratch_types=[
            pltpu.VMEM((CH,), jnp.int32),
            pltpu.VMEM((CH, D_IN), jnp.float32),
            pltpu.SemaphoreType.DMA,
        ],
    )
    def k(xn_hbm, rm_hbm, out_hbm, idx_v, rows_v, sem):
        wid = lax.axis_index("s") * 2 + lax.axis_index("c")
        base = wid * rpw

        def body(c, carry):
            off = pl.multiple_of(base + c * CH, 8)
            pltpu.sync_copy(rm_hbm.at[pl.ds(off, CH)], idx_v)
            pltpu.async_copy(xn_hbm.at[idx_v], rows_v, sem).wait()
            pltpu.sync_copy(rows_v, out_hbm.at[pl.ds(off, CH)])
            return carry

        lax.fori_loop(0, rpw // CH, body, 0)

    return k(xn, row_map)


def _gmm_body(tg_ref, vd_ref, x_ref, w1_ref, b1_ref, w2_ref, b2_ref, wp_ref,
              o_ref):
    i = pl.program_id(0)

    @pl.when(vd_ref[i] != 0)
    def _():
        xb = x_ref[...]
        h = lax.dot_general(xb, w1_ref[0], (((1,), (1,)), ((), ())),
                            preferred_element_type=jnp.float32) + b1_ref[0]
        h = _gelu(h)
        o = lax.dot_general(h, w2_ref[0], (((1,), (1,)), ((), ())),
                            preferred_element_type=jnp.float32) + b2_ref[0]
        wp = wp_ref[0].reshape(BM, 1)
        o_ref[...] = o * wp


def _grouped_expert_mlp(xpad, We1, be1, We2, be2, w_pad, tile_gid, valid):
    grid_spec = pltpu.PrefetchScalarGridSpec(
        num_scalar_prefetch=2,
        grid=(NT,),
        in_specs=[
            pl.BlockSpec((BM, D_IN), lambda i, tg, vd: (i, 0)),
            pl.BlockSpec((1, D_H, D_IN), lambda i, tg, vd: (tg[i], 0, 0)),
            pl.BlockSpec((1, 1, D_H), lambda i, tg, vd: (tg[i], 0, 0)),
            pl.BlockSpec((1, D_OUT, D_H), lambda i, tg, vd: (tg[i], 0, 0)),
            pl.BlockSpec((1, 1, D_OUT), lambda i, tg, vd: (tg[i], 0, 0)),
            pl.BlockSpec((1, 1, BM), lambda i, tg, vd: (i, 0, 0)),
        ],
        out_specs=pl.BlockSpec((BM, D_OUT), lambda i, tg, vd: (i, 0)),
    )
    return pl.pallas_call(
        _gmm_body,
        grid_spec=grid_spec,
        out_shape=jax.ShapeDtypeStruct((P, D_OUT), jnp.float32),
    )(tile_gid, valid, xpad,
      We1, be1.reshape(NUM_EXPERTS, 1, D_H),
      We2, be2.reshape(NUM_EXPERTS, 1, D_OUT),
      w_pad.reshape(NT, 1, BM))


def _sc_combine(S, Ew, p0, p1):
    ntok = S.shape[0]
    CHT = 16
    tpw = ntok // NW
    mesh = plsc.VectorSubcoreMesh(core_axis_name="c", subcore_axis_name="s")

    @functools.partial(
        pl.kernel, mesh=mesh,
        out_type=jax.ShapeDtypeStruct((ntok, D_OUT), jnp.float32),
        scratch_types=[
            pltpu.VMEM((CHT,), jnp.int32),
            pltpu.VMEM((CHT,), jnp.int32),
            pltpu.VMEM((CHT, D_OUT), jnp.float32),
            pltpu.VMEM((CHT, D_OUT), jnp.float32),
            pltpu.SemaphoreType.DMA,
        ],
    )
    def k(s_hbm, ew_hbm, p0_hbm, p1_hbm, out_hbm, i0_v, i1_v, acc_v, ebuf_v,
          sem):
        wid = lax.axis_index("s") * 2 + lax.axis_index("c")
        base = wid * tpw

        def accum():
            unroll = 8
            for r in range(CHT):
                def body(j, carry, r=r):
                    for u in range(unroll):
                        col = (j * unroll + u) * 16
                        acc_v[r, pl.ds(col, 16)] = (
                            acc_v[r, pl.ds(col, 16)]
                            + ebuf_v[r, pl.ds(col, 16)])
                    return carry
                lax.fori_loop(0, D_OUT // 16 // unroll, body, 0)

        def chunk(c, carry):
            off = pl.multiple_of(base + c * CHT, 8)
            pltpu.sync_copy(s_hbm.at[pl.ds(off, CHT)], acc_v)
            pltpu.sync_copy(p0_hbm.at[pl.ds(off, CHT)], i0_v)
            pltpu.sync_copy(p1_hbm.at[pl.ds(off, CHT)], i1_v)
            pltpu.async_copy(ew_hbm.at[i0_v], ebuf_v, sem).wait()
            accum()
            pltpu.async_copy(ew_hbm.at[i1_v], ebuf_v, sem).wait()
            accum()
            pltpu.sync_copy(acc_v, out_hbm.at[pl.ds(off, CHT)])
            return carry

        lax.fori_loop(0, tpw // CHT, chunk, 0)

    return k(S, Ew, p0, p1)


def _dispatch_plan(ti, tw):
    npair = ti.shape[0] * 2
    e = ti.reshape(npair)
    w = tw.reshape(npair)
    order = jnp.argsort(e, stable=True).astype(jnp.int32)
    es = e[order]
    counts = jnp.sum(e[None, :] == jnp.arange(NUM_EXPERTS)[:, None], axis=1)
    tiles_per = (counts + BM - 1) // BM
    tile_start = jnp.concatenate(
        [jnp.zeros((1,), jnp.int32), jnp.cumsum(tiles_per)[:-1].astype(jnp.int32)])
    group_start = jnp.concatenate(
        [jnp.zeros((1,), jnp.int32), jnp.cumsum(counts)[:-1].astype(jnp.int32)])
    used_tiles = jnp.sum(tiles_per)
    r = jnp.arange(npair, dtype=jnp.int32) - group_start[es]
    pos = tile_start[es] * BM + r
    row_map = jnp.zeros((P,), jnp.int32).at[pos].set(order // 2)
    w_pad = jnp.zeros((P,), jnp.float32).at[pos].set(w[order])
    ppos = jnp.zeros((npair,), jnp.int32).at[order].set(pos)
    p0 = ppos[0::2]
    p1 = ppos[1::2]
    tile_gid = (jnp.searchsorted(tile_start, jnp.arange(NT, dtype=jnp.int32),
                                 side="right") - 1).astype(jnp.int32)
    valid = (jnp.arange(NT) < used_tiles).astype(jnp.int32)
    return row_map, w_pad, p0, p1, tile_gid, valid


def kernel(x, norm_w, router_w, We1, be1, We2, be2, Ws1, bs1, Ws2, bs2):
    batch, seq, dim = x.shape
    out_len = (seq - K_POOL) // K_POOL + 1
    flat = x[:, :out_len * K_POOL, :].reshape(batch * out_len, dim * K_POOL)

    xn, ti, tw, S = _norm_router_shared(flat, norm_w, router_w, Ws1, bs1,
                                        Ws2, bs2)
    row_map, w_pad, p0, p1, tile_gid, valid = _dispatch_plan(ti, tw)
    xpad = _sc_gather_rows(xn, row_map)
    Ew = _grouped_expert_mlp(xpad, We1, be1, We2, be2, w_pad, tile_gid, valid)
    out = _sc_combine(S, Ew, p0, p1)
    return out.reshape(batch, out_len, D_OUT)
